# baseline (speedup 1.0000x reference)
import functools

import jax
import jax.numpy as jnp
from jax import lax
from jax.experimental import pallas as pl
from jax.experimental.pallas import tpu as pltpu
from jax.experimental.pallas import tpu_sc as plsc

_B = 16384
_D = 64
_NC = 2
_NS = 16
_NW = _NC * _NS
_BPW = _B // _NW
_CHUNK = 128
_NCHUNK = _BPW // _CHUNK


def _gather_body(users_hbm, isbns_hbm, ut_hbm, it_hbm, ue_out, ie_out,
                 uidx_v, iidx_v, urows_v, irows_v, sem):
    wid = lax.axis_index("s") * _NC + lax.axis_index("c")
    pltpu.sync_copy(users_hbm.at[wid], uidx_v)
    pltpu.sync_copy(isbns_hbm.at[wid], iidx_v)
    copies = []
    for j in range(_NCHUNK):
        copies.append(pltpu.async_copy(ut_hbm.at[uidx_v.at[j]], urows_v.at[j], sem))
        copies.append(pltpu.async_copy(it_hbm.at[iidx_v.at[j]], irows_v.at[j], sem))
    for c in copies:
        c.wait()
    pltpu.sync_copy(urows_v, ue_out.at[wid])
    pltpu.sync_copy(irows_v, ie_out.at[wid])


def _sc_gather(users, isbns, user_table, isbn_table):
    mesh = plsc.VectorSubcoreMesh(core_axis_name="c", subcore_axis_name="s")
    k = functools.partial(
        pl.kernel,
        mesh=mesh,
        out_type=(
            jax.ShapeDtypeStruct((_NW, _NCHUNK, _CHUNK, _D), jnp.float32),
            jax.ShapeDtypeStruct((_NW, _NCHUNK, _CHUNK, _D), jnp.float32),
        ),
        scratch_types=[
            pltpu.VMEM((_NCHUNK, _CHUNK), jnp.int32),
            pltpu.VMEM((_NCHUNK, _CHUNK), jnp.int32),
            pltpu.VMEM((_NCHUNK, _CHUNK, _D), jnp.float32),
            pltpu.VMEM((_NCHUNK, _CHUNK, _D), jnp.float32),
            pltpu.SemaphoreType.DMA,
        ],
        compiler_params=pltpu.CompilerParams(use_tc_tiling_on_sc=False),
    )(_gather_body)
    users_r = users.reshape(_NW, _NCHUNK, _CHUNK)
    isbns_r = isbns.reshape(_NW, _NCHUNK, _CHUNK)
    ue, ie = k(users_r, isbns_r, user_table, isbn_table)
    return ue.reshape(_B, _D), ie.reshape(_B, _D)


_BLK = 2048


def _mlp_body(ue_ref, ie_ref, w1_ref, b1_ref, w2_ref, b2_ref, o_ref):
    u = ue_ref[...]
    i = ie_ref[...]
    w1 = w1_ref[...]
    h = lax.dot_general(u, w1[:, :_D], (((1,), (1,)), ((), ())),
                        preferred_element_type=jnp.float32)
    h = h + lax.dot_general(i, w1[:, _D:], (((1,), (1,)), ((), ())),
                            preferred_element_type=jnp.float32)
    h = jnp.maximum(h + b1_ref[...], 0.0)
    o = lax.dot_general(w2_ref[...], h, (((1,), (1,)), ((), ())),
                        preferred_element_type=jnp.float32)
    o_ref[...] = o + b2_ref[0]


def _tc_mlp(ue, ie, W1, b1, W2, b2):
    hidden = W1.shape[0]
    grid = _B // _BLK
    return pl.pallas_call(
        _mlp_body,
        grid=(grid,),
        in_specs=[
            pl.BlockSpec((_BLK, _D), lambda g: (g, 0)),
            pl.BlockSpec((_BLK, _D), lambda g: (g, 0)),
            pl.BlockSpec((hidden, 2 * _D), lambda g: (0, 0)),
            pl.BlockSpec((1, hidden), lambda g: (0, 0)),
            pl.BlockSpec((1, hidden), lambda g: (0, 0)),
            pl.BlockSpec(memory_space=pltpu.SMEM),
        ],
        out_specs=pl.BlockSpec((1, _BLK), lambda g: (0, g)),
        out_shape=jax.ShapeDtypeStruct((1, _B), jnp.float32),
    )(ue, ie, W1, b1.reshape(1, hidden), W2, b2).reshape(_B, 1)


def kernel(users, isbns, user_table, isbn_table, W1, b1, W2, b2):
    ue, ie = _sc_gather(users, isbns, user_table, isbn_table)
    return _tc_mlp(ue, ie, W1, b1, W2, b2)
